# baseline (speedup 1.0000x reference)
import functools

import jax
import jax.numpy as jnp
from jax import lax
from jax.experimental import pallas as pl
from jax.experimental.pallas import tpu as pltpu
from jax.experimental.pallas import tpu_sc as plsc

B = 4096
D_USER, D_ITEM, D_CAT = 64, 64, 32
N_USER, N_ITEM, N_CAT = 1000000, 100000, 1000

NC, NS = 2, 16
NW = NC * NS
B_PER_W = B // NW
L = 16
GRP = B_PER_W // L
RING = 10
PRE = L - RING


@functools.lru_cache(maxsize=1)
def _build():
    mesh = plsc.VectorSubcoreMesh(core_axis_name="c", subcore_axis_name="s")

    @functools.partial(
        pl.kernel,
        mesh=mesh,
        compiler_params=pltpu.CompilerParams(needs_layout_passes=False),
        out_type=(
            jax.ShapeDtypeStruct((B, D_USER), jnp.float32),
            jax.ShapeDtypeStruct((B, D_ITEM), jnp.float32),
            jax.ShapeDtypeStruct((B, D_CAT), jnp.float32),
        ),
        scratch_types=[
            pltpu.VMEM((B_PER_W,), jnp.int32),
            pltpu.VMEM((B_PER_W,), jnp.int32),
            pltpu.VMEM((B_PER_W,), jnp.int32),
            pltpu.VMEM((RING, 64, 128), jnp.float32),
            pltpu.VMEM((32, N_CAT), jnp.float32),
            pltpu.VMEM((2, L, D_USER), jnp.float32),
            pltpu.VMEM((2, L, D_CAT), jnp.float32),
            pltpu.SemaphoreType.DMA,
            pltpu.SemaphoreType.DMA,
            pltpu.SemaphoreType.DMA,
        ],
    )
    def k(uid_hbm, iid_hbm, cid_hbm, wuT_hbm, wiT_hbm, wcT_hbm,
          ou_hbm, oi_hbm, oc_hbm,
          idx_v, lane_v, col_v, blk, catb, o16, o16c, s_tbl, s_out, s_cat):
        wid = lax.axis_index("s") * NC + lax.axis_index("c")
        base = wid * B_PER_W
        iota = lax.iota(jnp.int32, L)

        cat_stage = pltpu.async_copy(wcT_hbm, catb, s_cat)

        def sel(slot, lane, oslot, orow):
            lanes = jnp.full((L,), lane, jnp.int32)
            for q in range(4):
                vals = plsc.load_gather(blk.at[slot], [iota + q * L, lanes])
                o16[oslot, orow, pl.ds(q * L, L)] = vals

        def field(id_hbm, wT_hbm, o_hbm):
            pltpu.sync_copy(id_hbm.at[pl.ds(base, B_PER_W)], idx_v)
            def pre(g, _):
                ids = idx_v[pl.ds(g * L, L)]
                lanes = lax.bitwise_and(ids, 127)
                lane_v[pl.ds(g * L, L)] = lanes
                col_v[pl.ds(g * L, L)] = ids - lanes
                return 0
            lax.fori_loop(0, GRP, pre, 0)

            def wait_one():
                pltpu.make_async_copy(
                    wT_hbm.at[:, pl.ds(0, 128)], blk.at[0], s_tbl).wait()

            def grp(g, l_prev):
                l16 = lane_v[pl.ds(g * L, L)]
                c16 = col_v[pl.ds(g * L, L)]

                for j in range(L):
                    slot = lax.rem(g * L + j, RING)
                    if j == RING:
                        @pl.when(g >= 2)
                        def _():
                            pltpu.make_async_copy(
                                o_hbm.at[pl.ds(0, L)], o16.at[0], s_out).wait()

                        @pl.when(g >= 1)
                        def _():
                            pltpu.async_copy(
                                o16.at[lax.rem(g + 1, 2)],
                                o_hbm.at[pl.ds(base + (g - 1) * L, L)], s_out)
                    if j < RING:
                        @pl.when(g >= 1)
                        def _():
                            wait_one()
                            sel(slot, l_prev[j + PRE], lax.rem(g + 1, 2), j + PRE)
                    else:
                        wait_one()
                        sel(slot, l16[j - RING], lax.rem(g, 2), j - RING)
                    c0 = pl.multiple_of(c16[j], 128)
                    pltpu.async_copy(
                        wT_hbm.at[:, pl.ds(c0, 128)], blk.at[slot], s_tbl)
                return l16

            l_last = lax.fori_loop(0, GRP, grp, jnp.zeros((L,), jnp.int32))
            for j in range(RING):
                s = B_PER_W - RING + j
                wait_one()
                sel(s % RING, l_last[j + PRE], (GRP - 1) % 2, j + PRE)
            pltpu.async_copy(
                o16.at[(GRP - 1) % 2],
                o_hbm.at[pl.ds(base + (GRP - 1) * L, L)], s_out)
            for _ in range(2):
                pltpu.make_async_copy(
                    o_hbm.at[pl.ds(0, L)], o16.at[0], s_out).wait()

        field(uid_hbm, wuT_hbm, ou_hbm)
        field(iid_hbm, wiT_hbm, oi_hbm)

        cat_stage.wait()
        pltpu.sync_copy(cid_hbm.at[pl.ds(base, B_PER_W)], idx_v)

        def cgrp(g, _):
            ids = idx_v[pl.ds(g * L, L)]
            oslot = lax.rem(g, 2)

            @pl.when(g >= 2)
            def _():
                pltpu.make_async_copy(
                    oc_hbm.at[pl.ds(0, L)], o16c.at[0], s_out).wait()

            for j in range(L):
                cids = jnp.full((L,), ids[j], jnp.int32)
                for q in range(2):
                    vals = plsc.load_gather(catb, [iota + q * L, cids])
                    o16c[oslot, j, pl.ds(q * L, L)] = vals
            pltpu.async_copy(
                o16c.at[oslot], oc_hbm.at[pl.ds(base + g * L, L)], s_out)
            return 0

        lax.fori_loop(0, GRP, cgrp, 0)
        for _ in range(2):
            pltpu.make_async_copy(
                oc_hbm.at[pl.ds(0, L)], o16c.at[0], s_out).wait()

    return k


def kernel(user_id, item_id, category, W_user, W_item, W_cat):
    k = _build()
    ou, oi, oc = k(user_id.astype(jnp.int32), item_id.astype(jnp.int32),
                   category.astype(jnp.int32), W_user.T, W_item.T, W_cat.T)
    return jnp.concatenate([ou, oi, oc], axis=-1)


# speedup vs baseline: 1.0159x; 1.0159x over previous
import functools

import jax
import jax.numpy as jnp
from jax import lax
from jax.experimental import pallas as pl
from jax.experimental.pallas import tpu as pltpu
from jax.experimental.pallas import tpu_sc as plsc

B = 4096
D_USER, D_ITEM, D_CAT = 64, 64, 32
N_USER, N_ITEM, N_CAT = 1000000, 100000, 1000

NC, NS = 2, 16
NW = NC * NS
B_PER_W = B // NW
L = 16
GRP = B_PER_W // L
RING = 8


@functools.lru_cache(maxsize=1)
def _build():
    mesh = plsc.VectorSubcoreMesh(core_axis_name="c", subcore_axis_name="s")

    @functools.partial(
        pl.kernel,
        mesh=mesh,
        compiler_params=pltpu.CompilerParams(needs_layout_passes=False),
        out_type=(
            jax.ShapeDtypeStruct((B, D_USER), jnp.float32),
            jax.ShapeDtypeStruct((B, D_ITEM), jnp.float32),
            jax.ShapeDtypeStruct((B, D_CAT), jnp.float32),
        ),
        scratch_types=[
            pltpu.VMEM((B_PER_W,), jnp.int32),
            pltpu.VMEM((B_PER_W,), jnp.int32),
            pltpu.VMEM((B_PER_W,), jnp.int32),
            pltpu.VMEM((RING, 64, 128), jnp.float32),
            pltpu.VMEM((32, N_CAT), jnp.float32),
            pltpu.VMEM((2, L, D_USER), jnp.float32),
            pltpu.VMEM((2, L, D_CAT), jnp.float32),
            pltpu.SemaphoreType.DMA,
            pltpu.SemaphoreType.DMA,
            pltpu.SemaphoreType.DMA,
        ],
    )
    def k(uid_hbm, iid_hbm, cid_hbm, wuT_hbm, wiT_hbm, wcT_hbm,
          ou_hbm, oi_hbm, oc_hbm,
          idx_v, lane_v, col_v, blk, catb, o16, o16c, s_tbl, s_out, s_cat):
        wid = lax.axis_index("s") * NC + lax.axis_index("c")
        base = wid * B_PER_W
        iota = lax.iota(jnp.int32, L)

        cat_stage = pltpu.async_copy(wcT_hbm, catb, s_cat)

        def sel(slot, lane, oslot, orow):
            lanes = jnp.full((L,), lane, jnp.int32)
            for q in range(4):
                vals = plsc.load_gather(blk.at[slot], [iota + q * L, lanes])
                o16[oslot, orow, pl.ds(q * L, L)] = vals

        def field(id_hbm, wT_hbm, o_hbm):
            pltpu.sync_copy(id_hbm.at[pl.ds(base, B_PER_W)], idx_v)
            def pre(g, _):
                ids = idx_v[pl.ds(g * L, L)]
                lanes = lax.bitwise_and(ids, 127)
                lane_v[pl.ds(g * L, L)] = lanes
                col_v[pl.ds(g * L, L)] = ids - lanes
                return 0
            lax.fori_loop(0, GRP, pre, 0)

            def wait_one():
                pltpu.make_async_copy(
                    wT_hbm.at[:, pl.ds(0, 128)], blk.at[0], s_tbl).wait()

            def grp(g, l_prev):
                l16 = lane_v[pl.ds(g * L, L)]
                c16 = col_v[pl.ds(g * L, L)]

                def issue(j):
                    c0 = pl.multiple_of(c16[j], 128)
                    pltpu.async_copy(
                        wT_hbm.at[:, pl.ds(c0, 128)], blk.at[j % RING], s_tbl)

                for j in range(L):
                    if j == RING:
                        @pl.when(g >= 2)
                        def _():
                            pltpu.make_async_copy(
                                o_hbm.at[pl.ds(0, L)], o16.at[0], s_out).wait()

                        @pl.when(g >= 1)
                        def _():
                            pltpu.async_copy(
                                o16.at[lax.rem(g + 1, 2)],
                                o_hbm.at[pl.ds(base + (g - 1) * L, L)], s_out)
                    if j < RING:
                        @pl.when(g >= 1)
                        def _():
                            wait_one()
                            sel(j, l_prev[j + RING], lax.rem(g + 1, 2), j + RING)
                    else:
                        wait_one()
                        sel(j - RING, l16[j - RING], lax.rem(g, 2), j - RING)
                    issue(j)
                return l16

            l_last = lax.fori_loop(0, GRP, grp, jnp.zeros((L,), jnp.int32))
            for j in range(RING):
                wait_one()
                sel(j, l_last[j + RING], (GRP - 1) % 2, j + RING)
            pltpu.async_copy(
                o16.at[(GRP - 1) % 2],
                o_hbm.at[pl.ds(base + (GRP - 1) * L, L)], s_out)
            for _ in range(2):
                pltpu.make_async_copy(
                    o_hbm.at[pl.ds(0, L)], o16.at[0], s_out).wait()

        field(uid_hbm, wuT_hbm, ou_hbm)
        field(iid_hbm, wiT_hbm, oi_hbm)

        cat_stage.wait()
        pltpu.sync_copy(cid_hbm.at[pl.ds(base, B_PER_W)], idx_v)

        def cgrp(g, _):
            ids = idx_v[pl.ds(g * L, L)]
            oslot = lax.rem(g, 2)

            @pl.when(g >= 2)
            def _():
                pltpu.make_async_copy(
                    oc_hbm.at[pl.ds(0, L)], o16c.at[0], s_out).wait()

            for j in range(L):
                cids = jnp.full((L,), ids[j], jnp.int32)
                for q in range(2):
                    vals = plsc.load_gather(catb, [iota + q * L, cids])
                    o16c[oslot, j, pl.ds(q * L, L)] = vals
            pltpu.async_copy(
                o16c.at[oslot], oc_hbm.at[pl.ds(base + g * L, L)], s_out)
            return 0

        lax.fori_loop(0, GRP, cgrp, 0)
        for _ in range(2):
            pltpu.make_async_copy(
                oc_hbm.at[pl.ds(0, L)], o16c.at[0], s_out).wait()

    return k


def kernel(user_id, item_id, category, W_user, W_item, W_cat):
    k = _build()
    ou, oi, oc = k(user_id.astype(jnp.int32), item_id.astype(jnp.int32),
                   category.astype(jnp.int32), W_user.T, W_item.T, W_cat.T)
    return jnp.concatenate([ou, oi, oc], axis=-1)
